# baseline (speedup 1.0000x reference)
import jax
import jax.numpy as jnp
from jax.experimental import pallas as pl
from jax.experimental.pallas import tpu as pltpu

N = 5000
D = 12544
H = 1024
NC = 4
NB = 12
OW = 128

BN = 200
NN = N // BN
NCH = 28
CH = D // NCH
NSTEPS = NCH + NN
LOOK = 2
NBUF = LOOK + 1


def _copy(x_hbm, xbuf, sems, tile):
    t = jnp.clip(tile, 0, NN - 1)
    return pltpu.make_async_copy(
        x_hbm.at[pl.ds(t * BN, BN), :],
        xbuf.at[t % NBUF],
        sems.at[t % NBUF],
    )


def _body(x_hbm, w1_ref, w2_ref, b1_ref, b2_ref, w34_ref, b34_ref,
          out_ref, w1b_ref, xbuf, sems):
    s = pl.program_id(0)

    @pl.when(s < LOOK)
    def _prefetch():
        _copy(x_hbm, xbuf, sems, s).start()

    @pl.when(s < NCH)
    def _cast():
        w1b_ref[pl.ds(jnp.minimum(s, NCH - 1) * CH, CH), :] = (
            w1_ref[...].astype(jnp.bfloat16))

    @pl.when(s >= NCH)
    def _compute():
        tile = s - NCH
        @pl.when(tile + LOOK <= NN - 1)
        def _ahead():
            _copy(x_hbm, xbuf, sems, tile + LOOK).start()

        _copy(x_hbm, xbuf, sems, tile).wait()
        xb = xbuf[tile % NBUF]
        h1 = jax.lax.dot_general(
            xb, w1b_ref[...], (((1,), (0,)), ((), ())),
            preferred_element_type=jnp.float32)
        h1 = jnp.maximum(h1 + b1_ref[...], 0.0)
        h2 = jax.lax.dot_general(
            h1, w2_ref[...], (((1,), (0,)), ((), ())),
            preferred_element_type=jnp.float32) + b2_ref[...]
        h2 = jnp.maximum(h2, 0.0)
        o = jax.lax.dot_general(
            h2, w34_ref[...], (((1,), (0,)), ((), ())),
            preferred_element_type=jnp.float32) + b34_ref[...]
        col = jax.lax.broadcasted_iota(jnp.int32, o.shape, 1)
        is_cls = col < NC
        neg = jnp.where(is_cls, o, -1e30)
        m = jnp.max(neg, axis=1, keepdims=True)
        e = jnp.where(is_cls, jnp.exp(o - m), 0.0)
        sm = jnp.sum(e, axis=1, keepdims=True)
        out_ref[...] = jnp.where(is_cls, e / sm, o)


def kernel(feature_vectors, W1, b1, W2, b2, W3, b3, W4, b4):
    f32, bf16 = jnp.float32, jnp.bfloat16
    W34 = jnp.zeros((H, OW), f32).at[:, :NC].set(W3).at[:, NC:NC + NB].set(W4)
    b34 = jnp.zeros((1, OW), f32).at[0, :NC].set(b3).at[0, NC:NC + NB].set(b4)

    out = pl.pallas_call(
        _body,
        grid=(NSTEPS,),
        in_specs=[
            pl.BlockSpec(memory_space=pl.ANY),
            pl.BlockSpec((CH, H), lambda s: (jnp.minimum(s, NCH - 1), 0)),
            pl.BlockSpec((H, H), lambda s: (0, 0)),
            pl.BlockSpec((1, H), lambda s: (0, 0)),
            pl.BlockSpec((1, H), lambda s: (0, 0)),
            pl.BlockSpec((H, OW), lambda s: (0, 0)),
            pl.BlockSpec((1, OW), lambda s: (0, 0)),
        ],
        out_specs=pl.BlockSpec((BN, OW),
                               lambda s: (jnp.clip(s - NCH, 0, NN - 1), 0)),
        out_shape=jax.ShapeDtypeStruct((N, OW), f32),
        scratch_shapes=[
            pltpu.VMEM((D, H), bf16),
            pltpu.VMEM((NBUF, BN, D), f32),
            pltpu.SemaphoreType.DMA((NBUF,)),
        ],
        compiler_params=pltpu.CompilerParams(
            dimension_semantics=("arbitrary",),
            vmem_limit_bytes=62 * 1024 * 1024,
        ),
    )(feature_vectors, W1, W2.astype(bf16),
      b1.reshape(1, H), b2.reshape(1, H), W34.astype(bf16), b34)

    return out[:, :NC], out[:, NC:NC + NB]


# speedup vs baseline: 1.0838x; 1.0838x over previous
import jax
import jax.numpy as jnp
from jax.experimental import pallas as pl
from jax.experimental.pallas import tpu as pltpu

N = 5000
D = 12544
H = 1024
NC = 4
NB = 12
OW = 128

BN = 200
NN = N // BN
NCH = 8
CH = D // NCH
NSTEPS = NCH + NN


def _body(x_ref, w1_ref, w2_ref, b1_ref, b2_ref, w34_ref, b34_ref,
          out_ref, w1b_ref):
    s = pl.program_id(0)

    @pl.when(s < NCH)
    def _cast():
        w1b_ref[pl.ds(jnp.minimum(s, NCH - 1) * CH, CH), :] = (
            w1_ref[...].astype(jnp.bfloat16))

    @pl.when(s >= NCH)
    def _compute():
        h1 = jax.lax.dot_general(
            x_ref[...], w1b_ref[...], (((1,), (0,)), ((), ())),
            preferred_element_type=jnp.float32)
        h1 = jnp.maximum(h1 + b1_ref[...], 0.0)
        h2 = jax.lax.dot_general(
            h1, w2_ref[...], (((1,), (0,)), ((), ())),
            preferred_element_type=jnp.float32) + b2_ref[...]
        h2 = jnp.maximum(h2, 0.0)
        o = jax.lax.dot_general(
            h2, w34_ref[...], (((1,), (0,)), ((), ())),
            preferred_element_type=jnp.float32) + b34_ref[...]
        col = jax.lax.broadcasted_iota(jnp.int32, o.shape, 1)
        is_cls = col < NC
        neg = jnp.where(is_cls, o, -1e30)
        m = jnp.max(neg, axis=1, keepdims=True)
        e = jnp.where(is_cls, jnp.exp(o - m), 0.0)
        sm = jnp.sum(e, axis=1, keepdims=True)
        out_ref[...] = jnp.where(is_cls, e / sm, o)


def kernel(feature_vectors, W1, b1, W2, b2, W3, b3, W4, b4):
    f32, bf16 = jnp.float32, jnp.bfloat16
    W34 = jnp.zeros((H, OW), f32).at[:, :NC].set(W3).at[:, NC:NC + NB].set(W4)
    b34 = jnp.zeros((1, OW), f32).at[0, :NC].set(b3).at[0, NC:NC + NB].set(b4)

    out = pl.pallas_call(
        _body,
        grid=(NSTEPS,),
        in_specs=[
            pl.BlockSpec((BN, D),
                         lambda s: (jnp.clip(s - NCH, 0, NN - 1), 0)),
            pl.BlockSpec((CH, H),
                         lambda s: (jnp.minimum(s, NCH - 1), 0)),
            pl.BlockSpec((H, H), lambda s: (0, 0)),
            pl.BlockSpec((1, H), lambda s: (0, 0)),
            pl.BlockSpec((1, H), lambda s: (0, 0)),
            pl.BlockSpec((H, OW), lambda s: (0, 0)),
            pl.BlockSpec((1, OW), lambda s: (0, 0)),
        ],
        out_specs=pl.BlockSpec((BN, OW),
                               lambda s: (jnp.clip(s - NCH, 0, NN - 1), 0)),
        out_shape=jax.ShapeDtypeStruct((N, OW), f32),
        scratch_shapes=[pltpu.VMEM((D, H), bf16)],
        compiler_params=pltpu.CompilerParams(
            dimension_semantics=("arbitrary",),
            vmem_limit_bytes=62 * 1024 * 1024,
        ),
    )(feature_vectors, W1, W2.astype(bf16),
      b1.reshape(1, H), b2.reshape(1, H), W34.astype(bf16), b34)

    return out[:, :NC], out[:, NC:NC + NB]
